# Initial kernel scaffold; baseline (speedup 1.0000x reference)
import functools

import jax
import jax.numpy as jnp
from jax import lax
from jax.experimental import pallas as pl
from jax.experimental.pallas import tpu as pltpu
from jax.experimental.pallas import tpu_sc as plsc

N_NODES = 10000
D = 128

NC = 2
NS = 16
NW = NC * NS

C = 128
N_PAD = 10016
RPT = N_PAD // NS


def _sc_aggregate(feat, src_p, dst_p, zeros, n_chunk):
    e_t = n_chunk * C

    mesh = plsc.VectorSubcoreMesh(core_axis_name="c", subcore_axis_name="s")

    @functools.partial(
        pl.kernel,
        out_type=jax.ShapeDtypeStruct((NC * N_PAD, D), jnp.float32),
        mesh=mesh,
        scratch_types=[
            pltpu.VMEM((n_chunk, C), jnp.int32),
            pltpu.VMEM((n_chunk, C), jnp.int32),
            pltpu.VMEM((C, D), jnp.float32),
            pltpu.VMEM_SHARED((N_PAD, D), jnp.float32),
            pltpu.SemaphoreType.DMA,
        ],
    )
    def agg(feat_hbm, src_hbm, dst_hbm, zero_hbm, out_hbm,
            src_v, dst_v, rows_v, acc_sh, sem):
        c = lax.axis_index("c")
        s = lax.axis_index("s")
        row0 = s * RPT
        pltpu.sync_copy(zero_hbm.at[pl.ds(row0, RPT)], acc_sh.at[pl.ds(row0, RPT)])
        e0 = pl.multiple_of((c * NS + s) * e_t, 8)
        pltpu.sync_copy(src_hbm.at[pl.ds(e0, e_t)], src_v)
        pltpu.sync_copy(dst_hbm.at[pl.ds(e0, e_t)], dst_v)
        plsc.subcore_barrier()

        def step(i, carry):
            pltpu.async_copy(feat_hbm.at[src_v.at[i]], rows_v, sem).wait()
            pltpu.sync_copy(rows_v, acc_sh.at[dst_v.at[i]], add=True)
            return carry

        lax.fori_loop(0, n_chunk, step, 0)
        plsc.subcore_barrier()
        pltpu.sync_copy(
            acc_sh.at[pl.ds(row0, RPT)],
            out_hbm.at[pl.ds(c * N_PAD + row0, RPT)],
        )

    return agg(feat, src_p, dst_p, zeros)


def _tc_layer(partials, W, b, relu, bm=1252):
    n_blocks = N_PAD // bm

    def body(p0_ref, p1_ref, w_ref, b_ref, o_ref):
        t = p0_ref[...] + p1_ref[...]
        z = jnp.dot(t, w_ref[...], preferred_element_type=jnp.float32) + b_ref[...]
        if relu:
            z = jnp.maximum(z, 0.0)
        o_ref[...] = z

    return pl.pallas_call(
        body,
        grid=(n_blocks,),
        in_specs=[
            pl.BlockSpec((bm, D), lambda i: (i, 0)),
            pl.BlockSpec((bm, D), lambda i: (i + n_blocks, 0)),
            pl.BlockSpec((D, D), lambda i: (0, 0)),
            pl.BlockSpec((1, D), lambda i: (0, 0)),
        ],
        out_specs=pl.BlockSpec((bm, D), lambda i: (i, 0)),
        out_shape=jax.ShapeDtypeStruct((N_PAD, D), jnp.float32),
    )(partials, partials, W, b.reshape(1, D))


def kernel(x, edge_index, W1, b1, W2, b2):
    n_edges = edge_index.shape[1]
    src = edge_index[0].astype(jnp.int32)
    dst = edge_index[1].astype(jnp.int32)

    e_pad = ((n_edges + NW * C - 1) // (NW * C)) * (NW * C)
    pad = e_pad - n_edges
    if pad:
        src = jnp.concatenate([src, jnp.zeros((pad,), jnp.int32)])
        dst = jnp.concatenate([dst, jnp.full((pad,), N_NODES, jnp.int32)])
    n_chunk = e_pad // (NW * C)

    zeros = jnp.zeros((N_PAD, D), jnp.float32)

    p = _sc_aggregate(x, src, dst, zeros, n_chunk)
    h = _tc_layer(p, W1, b1, relu=True)
    q = _sc_aggregate(h, src, dst, zeros, n_chunk)
    out = _tc_layer(q, W2, b2, relu=False)
    return out[:N_NODES]


# speedup vs baseline: 2.8036x; 2.8036x over previous
import functools

import jax
import jax.numpy as jnp
from jax import lax
from jax.experimental import pallas as pl
from jax.experimental.pallas import tpu as pltpu
from jax.experimental.pallas import tpu_sc as plsc

N_NODES = 10000
D = 128

NC = 2
NS = 16
NW = NC * NS

C = 128
N_PAD = 10112
RPT = N_PAD // NS


def _sc_aggregate(feat, src_p, dst_p, zeros, n_chunk):
    mesh = plsc.VectorSubcoreMesh(core_axis_name="c", subcore_axis_name="s")

    @functools.partial(
        pl.kernel,
        out_type=jax.ShapeDtypeStruct((NC * N_PAD, D), jnp.float32),
        mesh=mesh,
        scratch_types=[
            pltpu.VMEM((n_chunk, C), jnp.int32),
            pltpu.VMEM((n_chunk, C), jnp.int32),
            pltpu.VMEM((C, D), jnp.float32),
            pltpu.VMEM_SHARED((N_PAD, D), jnp.float32),
            pltpu.SemaphoreType.DMA,
        ],
    )
    def agg(feat_hbm, src_hbm, dst_hbm, zero_hbm, out_hbm,
            src_v, dst_v, rows_v, acc_sh, sem):
        c = lax.axis_index("c")
        s = lax.axis_index("s")
        row0 = s * RPT
        pltpu.sync_copy(zero_hbm.at[pl.ds(row0, RPT)], acc_sh.at[pl.ds(row0, RPT)])
        k0 = (c * NS + s) * n_chunk
        pltpu.sync_copy(src_hbm.at[pl.ds(k0, n_chunk)], src_v)
        pltpu.sync_copy(dst_hbm.at[pl.ds(k0, n_chunk)], dst_v)
        plsc.subcore_barrier()

        def step(i, carry):
            pltpu.async_copy(feat_hbm.at[src_v.at[i]], rows_v, sem).wait()
            pltpu.sync_copy(rows_v, acc_sh.at[dst_v.at[i]], add=True)
            return carry

        lax.fori_loop(0, n_chunk, step, 0)
        plsc.subcore_barrier()
        pltpu.sync_copy(
            acc_sh.at[pl.ds(row0, RPT)],
            out_hbm.at[pl.ds(c * N_PAD + row0, RPT)],
        )

    return agg(feat, src_p, dst_p, zeros)


def _tc_layer(partials, W, b, relu, bm=2528):
    n_blocks = N_PAD // bm

    def body(p0_ref, p1_ref, w_ref, b_ref, o_ref):
        t = p0_ref[...] + p1_ref[...]
        z = jnp.dot(t, w_ref[...], preferred_element_type=jnp.float32) + b_ref[...]
        if relu:
            z = jnp.maximum(z, 0.0)
        o_ref[...] = z

    return pl.pallas_call(
        body,
        grid=(n_blocks,),
        in_specs=[
            pl.BlockSpec((bm, D), lambda i: (i, 0)),
            pl.BlockSpec((bm, D), lambda i: (i + n_blocks, 0)),
            pl.BlockSpec((D, D), lambda i: (0, 0)),
            pl.BlockSpec((1, D), lambda i: (0, 0)),
        ],
        out_specs=pl.BlockSpec((bm, D), lambda i: (i, 0)),
        out_shape=jax.ShapeDtypeStruct((N_PAD, D), jnp.float32),
    )(partials, partials, W, b.reshape(1, D))


def kernel(x, edge_index, W1, b1, W2, b2):
    n_edges = edge_index.shape[1]
    src = edge_index[0].astype(jnp.int32)
    dst = edge_index[1].astype(jnp.int32)

    quantum = NW * C * 8
    e_pad = ((n_edges + quantum - 1) // quantum) * quantum
    pad = e_pad - n_edges
    if pad:
        src = jnp.concatenate([src, jnp.zeros((pad,), jnp.int32)])
        dst = jnp.concatenate([dst, jnp.full((pad,), N_NODES, jnp.int32)])
    n_chunk = e_pad // (NW * C)
    src = src.reshape(NW * n_chunk, C)
    dst = dst.reshape(NW * n_chunk, C)

    zeros = jnp.zeros((N_PAD, D), jnp.float32)

    p = _sc_aggregate(x, src, dst, zeros, n_chunk)
    h = _tc_layer(p, W1, b1, relu=True)
    q = _sc_aggregate(h, src, dst, zeros, n_chunk)
    out = _tc_layer(q, W2, b2, relu=False)
    return out[:N_NODES]


# speedup vs baseline: 8.5562x; 3.0518x over previous
import functools

import jax
import jax.numpy as jnp
from jax import lax
from jax.experimental import pallas as pl
from jax.experimental.pallas import tpu as pltpu
from jax.experimental.pallas import tpu_sc as plsc

N_NODES = 10000
D = 128

NC = 2
NS = 16
NW = NC * NS

C = 128
N_PAD = 10112
RPT = N_PAD // NS


def _sc_aggregate(feat, src_p, dst_p, zeros, n_chunk, n_real_chunks):
    mesh = plsc.VectorSubcoreMesh(core_axis_name="c", subcore_axis_name="s")

    @functools.partial(
        pl.kernel,
        out_type=jax.ShapeDtypeStruct((NC * N_PAD, D), jnp.float32),
        mesh=mesh,
        scratch_types=[
            pltpu.VMEM((n_chunk, C), jnp.int32),
            pltpu.VMEM((n_chunk, C), jnp.int32),
            pltpu.VMEM((C, D), jnp.float32),
            pltpu.VMEM_SHARED((N_PAD, D), jnp.float32),
            pltpu.SemaphoreType.DMA,
        ],
    )
    def agg(feat_hbm, src_hbm, dst_hbm, zero_hbm, out_hbm,
            src_v, dst_v, rows_v, acc_sh, sem):
        c = lax.axis_index("c")
        s = lax.axis_index("s")
        row0 = s * RPT
        pltpu.sync_copy(zero_hbm.at[pl.ds(row0, RPT)], acc_sh.at[pl.ds(row0, RPT)])
        k0 = (c * NS + s) * n_chunk
        pltpu.sync_copy(src_hbm.at[pl.ds(k0, n_chunk)], src_v)
        pltpu.sync_copy(dst_hbm.at[pl.ds(k0, n_chunk)], dst_v)
        plsc.subcore_barrier()

        def step(i, carry):
            pltpu.async_copy(feat_hbm.at[src_v.at[i]], rows_v, sem).wait()
            pltpu.sync_copy(rows_v, acc_sh.at[dst_v.at[i]], add=True)
            return carry

        bound = jnp.clip(n_real_chunks - (c * NS + s) * n_chunk, 0, n_chunk)
        lax.fori_loop(0, bound, step, 0)
        plsc.subcore_barrier()
        pltpu.sync_copy(
            acc_sh.at[pl.ds(row0, RPT)],
            out_hbm.at[pl.ds(c * N_PAD + row0, RPT)],
        )

    return agg(feat, src_p, dst_p, zeros)


def _tc_layer(partials, W, b, relu, bm=2528):
    n_blocks = N_PAD // bm

    def body(p0_ref, p1_ref, w_ref, b_ref, o_ref):
        t = p0_ref[...] + p1_ref[...]
        z = jnp.dot(t, w_ref[...], preferred_element_type=jnp.float32) + b_ref[...]
        if relu:
            z = jnp.maximum(z, 0.0)
        o_ref[...] = z

    return pl.pallas_call(
        body,
        grid=(n_blocks,),
        in_specs=[
            pl.BlockSpec((bm, D), lambda i: (i, 0)),
            pl.BlockSpec((bm, D), lambda i: (i + n_blocks, 0)),
            pl.BlockSpec((D, D), lambda i: (0, 0)),
            pl.BlockSpec((1, D), lambda i: (0, 0)),
        ],
        out_specs=pl.BlockSpec((bm, D), lambda i: (i, 0)),
        out_shape=jax.ShapeDtypeStruct((N_PAD, D), jnp.float32),
    )(partials, partials, W, b.reshape(1, D))


def kernel(x, edge_index, W1, b1, W2, b2):
    n_edges = edge_index.shape[1]
    src = edge_index[0].astype(jnp.int32)
    dst = edge_index[1].astype(jnp.int32)

    quantum = NW * C * 8
    e_pad = ((n_edges + quantum - 1) // quantum) * quantum
    pad = e_pad - n_edges
    if pad:
        src = jnp.concatenate([src, jnp.zeros((pad,), jnp.int32)])
        pad_dst = N_NODES + (jnp.arange(pad, dtype=jnp.int32) % (N_PAD - N_NODES))
        dst = jnp.concatenate([dst, pad_dst])
    n_chunk = e_pad // (NW * C)
    n_real_chunks = (n_edges + C - 1) // C
    src = src.reshape(NW * n_chunk, C)
    dst = dst.reshape(NW * n_chunk, C)

    zeros = jnp.zeros((N_PAD, D), jnp.float32)

    p = _sc_aggregate(x, src, dst, zeros, n_chunk, n_real_chunks)
    h = _tc_layer(p, W1, b1, relu=True)
    q = _sc_aggregate(h, src, dst, zeros, n_chunk, n_real_chunks)
    out = _tc_layer(q, W2, b2, relu=False)
    return out[:N_NODES]


# speedup vs baseline: 14.2000x; 1.6596x over previous
import functools

import jax
import jax.numpy as jnp
from jax import lax
from jax.experimental import pallas as pl
from jax.experimental.pallas import tpu as pltpu
from jax.experimental.pallas import tpu_sc as plsc

N_NODES = 10000
D = 128

NC = 2
NS = 16
NW = NC * NS

C = 64
NBUF = 4
IBLK = 32
N_PAD = 10112
RPT = N_PAD // NS


def _sc_aggregate(feat, src_p, dst_p, n_chunk, n_real_chunks):
    n_iblk = n_chunk // IBLK
    assert n_chunk % IBLK == 0
    mesh = plsc.VectorSubcoreMesh(core_axis_name="c", subcore_axis_name="s")

    @functools.partial(
        pl.kernel,
        out_type=jax.ShapeDtypeStruct((NC * N_PAD, D), jnp.float32),
        mesh=mesh,
        scratch_types=[
            *[pltpu.VMEM((IBLK, C), jnp.int32) for _ in range(2)],
            *[pltpu.VMEM((IBLK, C), jnp.int32) for _ in range(2)],
            *[pltpu.VMEM((C, D), jnp.float32) for _ in range(NBUF)],
            pltpu.VMEM_SHARED((N_PAD, D), jnp.float32),
            *[pltpu.SemaphoreType.DMA for _ in range(2)],
            *[pltpu.SemaphoreType.DMA for _ in range(NBUF)],
        ],
    )
    def agg(feat_hbm, src_hbm, dst_hbm, out_hbm, *rest):
        sidx = rest[0:2]
        didx = rest[2:4]
        rows = rest[4:4 + NBUF]
        acc_sh = rest[4 + NBUF]
        isem = rest[5 + NBUF:7 + NBUF]
        gsem = rest[7 + NBUF:]
        c = lax.axis_index("c")
        s = lax.axis_index("s")
        row0 = s * RPT
        k0 = (c * NS + s) * n_chunk
        bound = jnp.clip(n_real_chunks - k0, 0, n_chunk)

        def stage_idx(j, jb):
            pltpu.async_copy(src_hbm.at[pl.ds(k0 + j * IBLK, IBLK)], sidx[jb], isem[jb])
            pltpu.async_copy(dst_hbm.at[pl.ds(k0 + j * IBLK, IBLK)], didx[jb], isem[jb])

        stage_idx(0, 0)
        if n_iblk > 1:
            stage_idx(1, 1)

        z16 = jnp.zeros((16,), jnp.float32)

        def zfill(r, carry):
            for l in range(D // 16):
                rows[0][r, pl.ds(l * 16, 16)] = z16
            return carry

        lax.fori_loop(0, C, zfill, 0)
        n_zcopy = RPT // C
        z_tail = RPT - n_zcopy * C
        for t in range(n_zcopy):
            pltpu.async_copy(rows[0], acc_sh.at[pl.ds(row0 + t * C, C)], gsem[0])
        if z_tail:
            pltpu.async_copy(
                rows[0].at[pl.ds(0, z_tail)],
                acc_sh.at[pl.ds(row0 + n_zcopy * C, z_tail)], gsem[0])
        for t in range(n_zcopy):
            pltpu.make_async_copy(
                rows[0], acc_sh.at[pl.ds(row0, C)], gsem[0]).wait()
        if z_tail:
            pltpu.make_async_copy(
                rows[0].at[pl.ds(0, z_tail)],
                acc_sh.at[pl.ds(row0, z_tail)], gsem[0]).wait()
        plsc.subcore_barrier()

        for j in range(n_iblk):
            jb = j % 2
            pltpu.make_async_copy(
                src_hbm.at[pl.ds(k0, IBLK)], sidx[jb], isem[jb]).wait()
            pltpu.make_async_copy(
                dst_hbm.at[pl.ds(k0, IBLK)], didx[jb], isem[jb]).wait()
            lb = jnp.clip(bound - j * IBLK, 0, IBLK)

            for b in range(NBUF):
                @pl.when(b < lb)
                def _(b=b):
                    pltpu.async_copy(feat_hbm.at[sidx[jb].at[b]], rows[b], gsem[b])

            def inner(g, carry):
                for b in range(NBUF):
                    k = g * NBUF + b

                    @pl.when(k < lb)
                    def _(b=b, k=k):
                        pltpu.make_async_copy(
                            feat_hbm.at[sidx[jb].at[k]], rows[b], gsem[b]).wait()
                        pltpu.sync_copy(rows[b], acc_sh.at[didx[jb].at[k]], add=True)

                        @pl.when(k + NBUF < lb)
                        def _():
                            pltpu.async_copy(
                                feat_hbm.at[sidx[jb].at[k + NBUF]], rows[b], gsem[b])
                return carry

            lax.fori_loop(0, (IBLK + NBUF - 1) // NBUF, inner, 0)
            if j + 2 < n_iblk:
                stage_idx(j + 2, jb)

        plsc.subcore_barrier()
        pltpu.sync_copy(
            acc_sh.at[pl.ds(row0, RPT)],
            out_hbm.at[pl.ds(c * N_PAD + row0, RPT)],
        )

    return agg(feat, src_p, dst_p)


def _tc_layer(partials, W, b, relu, bm=2000):
    n_blocks = N_NODES // bm

    def body(p0_ref, p1_ref, w_ref, b_ref, o_ref):
        t = p0_ref[0] + p1_ref[0]
        z = jnp.dot(t, w_ref[...], preferred_element_type=jnp.float32) + b_ref[...]
        if relu:
            z = jnp.maximum(z, 0.0)
        o_ref[...] = z

    return pl.pallas_call(
        body,
        grid=(n_blocks,),
        in_specs=[
            pl.BlockSpec((1, bm, D), lambda i: (0, i, 0)),
            pl.BlockSpec((1, bm, D), lambda i: (1, i, 0)),
            pl.BlockSpec((D, D), lambda i: (0, 0)),
            pl.BlockSpec((1, D), lambda i: (0, 0)),
        ],
        out_specs=pl.BlockSpec((bm, D), lambda i: (i, 0)),
        out_shape=jax.ShapeDtypeStruct((N_NODES, D), jnp.float32),
    )(partials, partials, W, b.reshape(1, D))


def kernel(x, edge_index, W1, b1, W2, b2):
    n_edges = edge_index.shape[1]
    src = edge_index[0].astype(jnp.int32)
    dst = edge_index[1].astype(jnp.int32)

    quantum = NW * C * 8
    e_pad = ((n_edges + quantum - 1) // quantum) * quantum
    pad = e_pad - n_edges
    if pad:
        src = jnp.concatenate([src, jnp.zeros((pad,), jnp.int32)])
        pad_dst = N_NODES + (jnp.arange(pad, dtype=jnp.int32) % (N_PAD - N_NODES))
        dst = jnp.concatenate([dst, pad_dst])
    n_chunk = e_pad // (NW * C)
    n_real_chunks = (n_edges + C - 1) // C
    src = src.reshape(NW * n_chunk, C)
    dst = dst.reshape(NW * n_chunk, C)

    p = _sc_aggregate(x, src, dst, n_chunk, n_real_chunks)
    h = _tc_layer(p.reshape(2, N_PAD, D), W1, b1, relu=True)
    q = _sc_aggregate(h, src, dst, n_chunk, n_real_chunks)
    return _tc_layer(q.reshape(2, N_PAD, D), W2, b2, relu=False)
